# baseline (speedup 1.0000x reference)
import functools

import jax
import jax.numpy as jnp
from jax import lax
from jax.experimental import pallas as pl
from jax.experimental.pallas import tpu as pltpu
from jax.experimental.pallas import tpu_sc as plsc

SEQ = 512
HALF = SEQ // 2
BATCH = 16
TOTAL = 32768
BUDGET = SEQ - 2
CLS_ID = 0
SEP_ID = 2
PAD_ID = 1
BUF = 272


def _body(tokens_hbm, cu_hbm, words_hbm, mask_hbm, type_hbm,
          cu_v, buf_v, row_v, mrow_v, zrow_v, sem_cu, sem_out):
    b = lax.axis_index("s")
    h = lax.axis_index("c")
    col0 = h * HALF

    cu_a = pltpu.async_copy(cu_hbm.at[pl.ds(0, 16)],
                            cu_v.at[pl.ds(0, 16)], sem_cu)
    cu_b = pltpu.async_copy(cu_hbm.at[pl.ds(16, 1)],
                            cu_v.at[pl.ds(16, 1)], sem_cu)

    zero = jnp.zeros((16,), jnp.int32)
    for c in range(HALF // 16):
        zrow_v[pl.ds(c * 16, 16)] = zero
    z_dma = pltpu.async_copy(
        zrow_v, type_hbm.at[b, pl.ds(pl.multiple_of(col0, 8), HALF)], sem_out)

    cu_a.wait()
    cu_b.wait()
    lane = lax.iota(jnp.int32, 16)
    cu_lo = cu_v[pl.ds(0, 16)]
    cu_top = cu_v[pl.ds(16, 16)]
    start = jnp.sum(jnp.where(lane == b, cu_lo, 0))
    end = (jnp.sum(jnp.where(lane == b + 1, cu_lo, 0))
           + jnp.sum(jnp.where((lane + 16) == b + 1, cu_top, 0)))
    seg_len = jnp.minimum(end - start, BUDGET)
    seg_sep = seg_len + 1

    base = start + col0
    base_al = pl.multiple_of(
        jnp.minimum((jnp.maximum(base - 8, 0) >> 3) << 3, TOTAL - BUF), 8)
    pltpu.sync_copy(tokens_hbm.at[pl.ds(base_al, BUF)], buf_v)
    off = base - base_al

    for c in range(HALF // 16):
        j = lane + (col0 + c * 16)
        gidx = jnp.clip(off + (c * 16) + lane - 1, 0, BUF - 1)
        tok = plsc.load_gather(buf_v, [gidx])
        words = jnp.where(
            j <= seg_len,
            tok,
            jnp.where(j == seg_sep, SEP_ID, PAD_ID),
        )
        if c == 0:
            words = jnp.where(j == 0, CLS_ID, words)
        row_v[pl.ds(c * 16, 16)] = words
        mrow_v[pl.ds(c * 16, 16)] = jnp.where(j <= seg_sep, 1, 0)

    col0_al = pl.multiple_of(col0, 8)
    w_dma = pltpu.async_copy(row_v, words_hbm.at[b, pl.ds(col0_al, HALF)],
                             sem_out)
    m_dma = pltpu.async_copy(mrow_v, mask_hbm.at[b, pl.ds(col0_al, HALF)],
                             sem_out)
    z_dma.wait()
    w_dma.wait()
    m_dma.wait()


@jax.jit
def _pack(tokens, cu_seqlens):
    out_t = jax.ShapeDtypeStruct((BATCH, SEQ), jnp.int32)
    k = pl.kernel(
        _body,
        out_type=(out_t, out_t, out_t),
        mesh=plsc.VectorSubcoreMesh(core_axis_name="c", subcore_axis_name="s"),
        scratch_types=[
            pltpu.VMEM((32,), jnp.int32),
            pltpu.VMEM((BUF,), jnp.int32),
            pltpu.VMEM((HALF,), jnp.int32),
            pltpu.VMEM((HALF,), jnp.int32),
            pltpu.VMEM((HALF,), jnp.int32),
            pltpu.SemaphoreType.DMA,
            pltpu.SemaphoreType.DMA,
        ],
        compiler_params=pltpu.CompilerParams(needs_layout_passes=False),
    )
    return k(tokens, cu_seqlens)


def kernel(tokens, cu_seqlens):
    return _pack(tokens, cu_seqlens)


# speedup vs baseline: 1.0590x; 1.0590x over previous
import functools

import jax
import jax.numpy as jnp
from jax import lax
from jax.experimental import pallas as pl
from jax.experimental.pallas import tpu as pltpu
from jax.experimental.pallas import tpu_sc as plsc

SEQ = 512
BATCH = 16
TOTAL = 32768
BUDGET = SEQ - 2
CLS_ID = 0
SEP_ID = 2
PAD_ID = 1
BUF = 528


def _body(tokens_hbm, cu_hbm, words_hbm, mask_hbm, type_hbm,
          cu_v, buf_v, row_v, mrow_v, zrow_v, sem_cu, sem_out):
    b = lax.axis_index("s")

    cu_a = pltpu.async_copy(cu_hbm.at[pl.ds(0, 16)],
                            cu_v.at[pl.ds(0, 16)], sem_cu)
    cu_b = pltpu.async_copy(cu_hbm.at[pl.ds(16, 1)],
                            cu_v.at[pl.ds(16, 1)], sem_cu)

    zero = jnp.zeros((16,), jnp.int32)
    for c in range(SEQ // 16):
        zrow_v[pl.ds(c * 16, 16)] = zero
    z_dma = pltpu.async_copy(zrow_v, type_hbm.at[b], sem_out)

    cu_a.wait()
    cu_b.wait()
    lane = lax.iota(jnp.int32, 16)
    cu_lo = cu_v[pl.ds(0, 16)]
    cu_top = cu_v[pl.ds(16, 16)]
    start = jnp.sum(jnp.where(lane == b, cu_lo, 0))
    end = (jnp.sum(jnp.where(lane == b + 1, cu_lo, 0))
           + jnp.sum(jnp.where((lane + 16) == b + 1, cu_top, 0)))
    seg_len = jnp.minimum(end - start, BUDGET)
    seg_sep = seg_len + 1

    start_al = pl.multiple_of(
        jnp.minimum((jnp.maximum(start - 8, 0) >> 3) << 3, TOTAL - BUF), 8)
    pltpu.sync_copy(tokens_hbm.at[pl.ds(start_al, BUF)], buf_v)
    off = start - start_al

    for c in range(SEQ // 16):
        j = lane + (c * 16)
        gidx = jnp.clip(off + (c * 16) + lane - 1, 0, BUF - 1)
        tok = plsc.load_gather(buf_v, [gidx])
        words = jnp.where(
            j <= seg_len,
            tok,
            jnp.where(j == seg_sep, SEP_ID, PAD_ID),
        )
        if c == 0:
            words = jnp.where(j == 0, CLS_ID, words)
        row_v[pl.ds(c * 16, 16)] = words
        mrow_v[pl.ds(c * 16, 16)] = jnp.where(j <= seg_sep, 1, 0)

    w_dma = pltpu.async_copy(row_v, words_hbm.at[b], sem_out)
    m_dma = pltpu.async_copy(mrow_v, mask_hbm.at[b], sem_out)
    z_dma.wait()
    w_dma.wait()
    m_dma.wait()


@jax.jit
def _pack(tokens, cu_seqlens):
    out_t = jax.ShapeDtypeStruct((BATCH, SEQ), jnp.int32)
    k = pl.kernel(
        _body,
        out_type=(out_t, out_t, out_t),
        mesh=plsc.VectorSubcoreMesh(core_axis_name="c", subcore_axis_name="s",
                                    num_cores=1),
        scratch_types=[
            pltpu.VMEM((32,), jnp.int32),
            pltpu.VMEM((BUF,), jnp.int32),
            pltpu.VMEM((SEQ,), jnp.int32),
            pltpu.VMEM((SEQ,), jnp.int32),
            pltpu.VMEM((SEQ,), jnp.int32),
            pltpu.SemaphoreType.DMA,
            pltpu.SemaphoreType.DMA,
        ],
        compiler_params=pltpu.CompilerParams(needs_layout_passes=False),
    )
    return k(tokens, cu_seqlens)


def kernel(tokens, cu_seqlens):
    return _pack(tokens, cu_seqlens)
